# baseline (speedup 1.0000x reference)
import functools

import jax
import jax.numpy as jnp
from jax import lax
from jax.experimental import pallas as pl
from jax.experimental.pallas import tpu as pltpu
from jax.experimental.pallas import tpu_sc as plsc

VOCAB = 100000
C = 64
ALPHA = 0.01
L = 200
B = 1024

V_BLK = 2048
V_PAD = 102400
CP = 128
NC, NS = 2, 16
NW = NC * NS
EPW = B // NW
LH = L // 2
NCHUNK = 2 * EPW


def _logtable_body(freq_ref, comm_ref, out_ref):
    i = pl.program_id(0)
    n = comm_ref[:, :]
    denom = n + VOCAB * ALPHA
    p = freq_ref[:, :] * n
    p = jnp.where(p == 0.0, ALPHA, p)
    lv = jnp.log(p / denom)
    lvt = lv.T
    v = i * V_BLK + lax.broadcasted_iota(jnp.int32, (V_BLK, C), 0)
    out_ref[:, pl.ds(0, C)] = jnp.where(v < VOCAB, lvt, 0.0)
    out_ref[:, pl.ds(C, CP - C)] = jnp.zeros((V_BLK, CP - C), jnp.float32)


def _midx_body(m_ref, lens_ref, out_ref):
    li = lax.broadcasted_iota(jnp.int32, (L, B), 0)
    mi = jnp.where(li < lens_ref[:, :], m_ref[:, :], VOCAB)
    out_ref[:, :] = mi.T


def _softmax_body(s_ref, out_ref):
    s = s_ref[:, pl.ds(0, C)]
    mx = jnp.max(s, axis=1, keepdims=True)
    e = jnp.exp(s - mx)
    out_ref[:, :] = e / jnp.sum(e, axis=1, keepdims=True)


_sc_mesh = plsc.VectorSubcoreMesh(core_axis_name="c", subcore_axis_name="s")


@functools.partial(
    pl.kernel,
    mesh=_sc_mesh,
    out_type=jax.ShapeDtypeStruct((B, CP), jnp.float32),
    scratch_types=[
        pltpu.VMEM((NCHUNK, LH), jnp.int32),
        pltpu.VMEM((EPW, LH), jnp.int32),
        pltpu.VMEM((LH, CP), jnp.float32),
        pltpu.VMEM_SHARED((B, CP), jnp.float32),
        pltpu.SemaphoreType.DMA,
    ],
)
def _sc_infer(midx_hbm, dst_hbm, zeros_hbm, logt_hbm, out_hbm,
              idx_v, dst_v, rows_v, acc_sh, sem):
    wid = lax.axis_index("s") * NC + lax.axis_index("c")
    base = wid * EPW
    pltpu.sync_copy(midx_hbm.at[pl.ds(base * 2, NCHUNK)], idx_v)
    pltpu.sync_copy(dst_hbm.at[pl.ds(base, EPW)], dst_v)
    pltpu.sync_copy(zeros_hbm, acc_sh.at[pl.ds(base, EPW)])

    def chunk(k, carry):
        e = k // 2
        pltpu.async_copy(logt_hbm.at[idx_v.at[k]], rows_v, sem).wait()
        pltpu.sync_copy(rows_v, acc_sh.at[dst_v.at[e]], add=True)
        return carry

    lax.fori_loop(0, NCHUNK, chunk, 0)
    pltpu.sync_copy(acc_sh.at[pl.ds(base, EPW)], out_hbm.at[pl.ds(base, EPW)])


def kernel(m, m_lens, unigram_freq, comm_N):
    m = m.astype(jnp.int32)
    m_lens = m_lens.astype(jnp.int32)
    fp = jnp.pad(unigram_freq, ((0, 0), (0, V_PAD - VOCAB)))
    logt = pl.pallas_call(
        _logtable_body,
        grid=(V_PAD // V_BLK,),
        in_specs=[
            pl.BlockSpec((C, V_BLK), lambda i: (0, i)),
            pl.BlockSpec((C, 1), lambda i: (0, 0)),
        ],
        out_specs=pl.BlockSpec((V_BLK, CP), lambda i: (i, 0)),
        out_shape=jax.ShapeDtypeStruct((V_PAD, CP), jnp.float32),
    )(fp, comm_N.reshape(C, 1))
    midx = pl.pallas_call(
        _midx_body,
        out_shape=jax.ShapeDtypeStruct((B, L), jnp.int32),
    )(m, m_lens.reshape(1, B))
    dst = jnp.broadcast_to(
        jnp.arange(B, dtype=jnp.int32)[:, None], (B, LH))
    zeros = jnp.zeros((EPW, CP), jnp.float32)
    sums = _sc_infer(midx.reshape(B * 2, LH), dst, zeros, logt)
    return pl.pallas_call(
        _softmax_body,
        out_shape=jax.ShapeDtypeStruct((B, C), jnp.float32),
    )(sums)
